# baseline (speedup 1.0000x reference)
import functools

import jax
import jax.numpy as jnp
from jax import lax
from jax.experimental import pallas as pl
from jax.experimental.pallas import tpu as pltpu
from jax.experimental.pallas import tpu_sc as plsc

BATCH = 16384
OUT_DIM = 64
ID_DIM = 32
TEXT_DIM = 128
WIDE = 128

_NC = 2
_NS = 16
_NW = _NC * _NS
_BPW = BATCH // _NW
_CHUNK = _BPW // 2


def _sc_gather(uidx_hbm, iidx_hbm, uemb_hbm, iemb_hbm, u_out, i_out,
               uidx_v, iidx_v, u_buf, i_buf, sem_u, sem_i):
    wid = lax.axis_index("s") * _NC + lax.axis_index("c")
    base = wid * _BPW
    pltpu.sync_copy(uidx_hbm.at[pl.ds(base, _BPW)], uidx_v)
    pltpu.sync_copy(iidx_hbm.at[pl.ds(base, _BPW)], iidx_v)
    for c in range(_BPW // _CHUNK):
        cp_u = pltpu.async_copy(
            uemb_hbm.at[uidx_v.at[pl.ds(c * _CHUNK, _CHUNK)]], u_buf, sem_u)
        cp_i = pltpu.async_copy(
            iemb_hbm.at[iidx_v.at[pl.ds(c * _CHUNK, _CHUNK)]], i_buf, sem_i)
        cp_u.wait()
        cp_i.wait()
        pltpu.sync_copy(u_buf, u_out.at[pl.ds(base + c * _CHUNK, _CHUNK)])
        pltpu.sync_copy(i_buf, i_out.at[pl.ds(base + c * _CHUNK, _CHUNK)])


@functools.cache
def _gather_call():
    return pl.kernel(
        _sc_gather,
        mesh=plsc.VectorSubcoreMesh(core_axis_name="c", subcore_axis_name="s"),
        out_type=(
            jax.ShapeDtypeStruct((BATCH, WIDE), jnp.float32),
            jax.ShapeDtypeStruct((BATCH, WIDE), jnp.float32),
        ),
        scratch_types=[
            pltpu.VMEM((_BPW,), jnp.int32),
            pltpu.VMEM((_BPW,), jnp.int32),
            pltpu.VMEM((_CHUNK, WIDE), jnp.float32),
            pltpu.VMEM((_CHUNK, WIDE), jnp.float32),
            pltpu.SemaphoreType.DMA,
            pltpu.SemaphoreType.DMA,
        ],
    )


_TC_ROWS = 512
_N_BLOCKS = BATCH // _TC_ROWS


def _tc_combine(x_ref, uw_ref, iw_ref, usel_ref, isel_ref, w_ref, b_ref,
                out_ref):
    t = jnp.dot(x_ref[...], w_ref[...], preferred_element_type=jnp.float32)
    t = t + b_ref[...]
    usel = usel_ref[...][:, None]
    u = jnp.where(usel == 0, uw_ref[:, :OUT_DIM], uw_ref[:, OUT_DIM:])
    isel = isel_ref[...][:, None]
    i01 = jnp.where(isel == 0, iw_ref[:, 0:32], iw_ref[:, 32:64])
    i23 = jnp.where(isel == 2, iw_ref[:, 64:96], iw_ref[:, 96:128])
    idv = jnp.where(isel < 2, i01, i23)
    s = jnp.sum(u[:, :ID_DIM] * idv, axis=1)
    s = s + jnp.sum(u[:, ID_DIM:] * t, axis=1)
    out_ref[...] = jax.nn.sigmoid(s)


def _combine(x, u_wide, i_wide, usel, isel, W_text, b2):
    return pl.pallas_call(
        _tc_combine,
        grid=(_N_BLOCKS,),
        in_specs=[
            pl.BlockSpec((_TC_ROWS, TEXT_DIM), lambda i: (i, 0)),
            pl.BlockSpec((_TC_ROWS, WIDE), lambda i: (i, 0)),
            pl.BlockSpec((_TC_ROWS, WIDE), lambda i: (i, 0)),
            pl.BlockSpec((_TC_ROWS,), lambda i: (i,)),
            pl.BlockSpec((_TC_ROWS,), lambda i: (i,)),
            pl.BlockSpec((TEXT_DIM, ID_DIM), lambda i: (0, 0)),
            pl.BlockSpec((1, ID_DIM), lambda i: (0, 0)),
        ],
        out_specs=pl.BlockSpec((_TC_ROWS,), lambda i: (i,)),
        out_shape=jax.ShapeDtypeStruct((BATCH,), jnp.float32),
    )(x, u_wide, i_wide, usel, isel, W_text, b2)


def kernel(user_ids, item_ids, item_text_feats, user_emb, item_id_emb,
           W_text, b_text):
    uw = user_emb.reshape(-1, WIDE)
    iw = item_id_emb.reshape(-1, WIDE)
    uw_idx = user_ids >> 1
    iw_idx = item_ids >> 2
    usel = user_ids & 1
    isel = item_ids & 3
    u_wide, i_wide = _gather_call()(uw_idx, iw_idx, uw, iw)
    return _combine(item_text_feats, u_wide, i_wide, usel, isel, W_text,
                    b_text.reshape(1, ID_DIM))


# speedup vs baseline: 1.4847x; 1.4847x over previous
import functools

import jax
import jax.numpy as jnp
from jax import lax
from jax.experimental import pallas as pl
from jax.experimental.pallas import tpu as pltpu
from jax.experimental.pallas import tpu_sc as plsc

BATCH = 16384
OUT_DIM = 64
ID_DIM = 32
TEXT_DIM = 128

_NC = 2
_NS = 16
_NW = _NC * _NS
_BPW = BATCH // _NW
_CH = 256
_K = 16


def _sc_gather(uids_hbm, iids_hbm, uemb_hbm, iemb_hbm, u_out, i_out,
               uid_s, iid_s, u_rows, i_rows, sem_u, sem_i):
    wid = lax.axis_index("s") * _NC + lax.axis_index("c")
    base = wid * _BPW
    pltpu.sync_copy(uids_hbm.at[pl.ds(base, _BPW)], uid_s)
    pltpu.sync_copy(iids_hbm.at[pl.ds(base, _BPW)], iid_s)

    for c in range(_BPW // _CH):

        def group_body(g, c=c):
            off = c * _CH + g * _K
            uvec = uid_s[pl.ds(off, _K)]
            ivec = iid_s[pl.ds(off, _K)]
            cps = []
            for j in range(_K):
                uid = uvec[j]
                iid = ivec[j]
                cps.append(pltpu.async_copy(
                    uemb_hbm.at[pl.ds(uid, 1)],
                    u_rows.at[pl.ds(g * _K + j, 1)], sem_u))
                cps.append(pltpu.async_copy(
                    iemb_hbm.at[pl.ds(iid, 1)],
                    i_rows.at[pl.ds(g * _K + j, 1)], sem_i))
            for cp in cps:
                cp.wait()

        pl.loop(0, _CH // _K)(group_body)
        pltpu.sync_copy(u_rows, u_out.at[pl.ds(base + c * _CH, _CH)])
        pltpu.sync_copy(i_rows, i_out.at[pl.ds(base + c * _CH, _CH)])


@functools.cache
def _gather_call():
    return pl.kernel(
        _sc_gather,
        mesh=plsc.VectorSubcoreMesh(core_axis_name="c", subcore_axis_name="s"),
        out_type=(
            jax.ShapeDtypeStruct((BATCH, OUT_DIM), jnp.float32),
            jax.ShapeDtypeStruct((BATCH, ID_DIM), jnp.float32),
        ),
        scratch_types=[
            pltpu.VMEM((_BPW,), jnp.int32),
            pltpu.VMEM((_BPW,), jnp.int32),
            pltpu.VMEM((_CH, OUT_DIM), jnp.float32),
            pltpu.VMEM((_CH, ID_DIM), jnp.float32),
            pltpu.SemaphoreType.DMA,
            pltpu.SemaphoreType.DMA,
        ],
    )


_TC_ROWS = 512
_N_BLOCKS = BATCH // _TC_ROWS


def _tc_combine(x_ref, u_ref, id_ref, w_ref, b_ref, out_ref):
    t = jnp.dot(x_ref[...], w_ref[...], preferred_element_type=jnp.float32)
    t = t + b_ref[...]
    s = jnp.sum(u_ref[:, :ID_DIM] * id_ref[...], axis=1)
    s = s + jnp.sum(u_ref[:, ID_DIM:] * t, axis=1)
    out_ref[...] = jax.nn.sigmoid(s)


def _combine(x, u_gath, i_gath, W_text, b2):
    return pl.pallas_call(
        _tc_combine,
        grid=(_N_BLOCKS,),
        in_specs=[
            pl.BlockSpec((_TC_ROWS, TEXT_DIM), lambda i: (i, 0)),
            pl.BlockSpec((_TC_ROWS, OUT_DIM), lambda i: (i, 0)),
            pl.BlockSpec((_TC_ROWS, ID_DIM), lambda i: (i, 0)),
            pl.BlockSpec((TEXT_DIM, ID_DIM), lambda i: (0, 0)),
            pl.BlockSpec((1, ID_DIM), lambda i: (0, 0)),
        ],
        out_specs=pl.BlockSpec((_TC_ROWS,), lambda i: (i,)),
        out_shape=jax.ShapeDtypeStruct((BATCH,), jnp.float32),
    )(x, u_gath, i_gath, W_text, b2)


def kernel(user_ids, item_ids, item_text_feats, user_emb, item_id_emb,
           W_text, b_text):
    u_gath, i_gath = _gather_call()(user_ids, item_ids, user_emb, item_id_emb)
    return _combine(item_text_feats, u_gath, i_gath, W_text,
                    b_text.reshape(1, ID_DIM))


# speedup vs baseline: 1.4958x; 1.0075x over previous
import functools

import jax
import jax.numpy as jnp
from jax import lax
from jax.experimental import pallas as pl
from jax.experimental.pallas import tpu as pltpu
from jax.experimental.pallas import tpu_sc as plsc

BATCH = 16384
OUT_DIM = 64
ID_DIM = 32
TEXT_DIM = 128

_NC = 2
_NS = 16
_NW = _NC * _NS
_BPW = BATCH // _NW
_CH = 256
_K = 16


def _sc_gather_one(ids_hbm, emb_hbm, out_hbm, id_s, rows, sem):
    wid = lax.axis_index("s") * _NC + lax.axis_index("c")
    base = wid * _BPW
    pltpu.sync_copy(ids_hbm.at[pl.ds(base, _BPW)], id_s)

    for c in range(_BPW // _CH):

        def group_body(g, c=c):
            off = c * _CH + g * _K
            vec = id_s[pl.ds(off, _K)]
            cps = []
            for j in range(_K):
                rid = vec[j]
                cps.append(pltpu.async_copy(
                    emb_hbm.at[pl.ds(rid, 1)],
                    rows.at[pl.ds(g * _K + j, 1)], sem))
            for cp in cps:
                cp.wait()

        pl.loop(0, _CH // _K)(group_body)
        pltpu.sync_copy(rows, out_hbm.at[pl.ds(base + c * _CH, _CH)])


@functools.cache
def _gather_call(dim):
    return pl.kernel(
        _sc_gather_one,
        mesh=plsc.VectorSubcoreMesh(core_axis_name="c", subcore_axis_name="s"),
        out_type=jax.ShapeDtypeStruct((BATCH, dim), jnp.float32),
        scratch_types=[
            pltpu.VMEM((_BPW,), jnp.int32),
            pltpu.VMEM((_CH, dim), jnp.float32),
            pltpu.SemaphoreType.DMA,
        ],
    )


_TC_ROWS = 512
_N_BLOCKS = BATCH // _TC_ROWS


def _tc_text(x_ref, w_ref, b_ref, out_ref):
    t = jnp.dot(x_ref[...], w_ref[...], preferred_element_type=jnp.float32)
    out_ref[...] = t + b_ref[...]


def _text_proj(x, W_text, b2):
    return pl.pallas_call(
        _tc_text,
        grid=(_N_BLOCKS,),
        in_specs=[
            pl.BlockSpec((_TC_ROWS, TEXT_DIM), lambda i: (i, 0)),
            pl.BlockSpec((TEXT_DIM, ID_DIM), lambda i: (0, 0)),
            pl.BlockSpec((1, ID_DIM), lambda i: (0, 0)),
        ],
        out_specs=pl.BlockSpec((_TC_ROWS, ID_DIM), lambda i: (i, 0)),
        out_shape=jax.ShapeDtypeStruct((BATCH, ID_DIM), jnp.float32),
    )(x, W_text, b2)


def _tc_combine(t_ref, u_ref, id_ref, out_ref):
    s = jnp.sum(u_ref[:, :ID_DIM] * id_ref[...], axis=1)
    s = s + jnp.sum(u_ref[:, ID_DIM:] * t_ref[...], axis=1)
    out_ref[...] = jax.nn.sigmoid(s)


def _combine(t, u_gath, i_gath):
    return pl.pallas_call(
        _tc_combine,
        grid=(_N_BLOCKS,),
        in_specs=[
            pl.BlockSpec((_TC_ROWS, ID_DIM), lambda i: (i, 0)),
            pl.BlockSpec((_TC_ROWS, OUT_DIM), lambda i: (i, 0)),
            pl.BlockSpec((_TC_ROWS, ID_DIM), lambda i: (i, 0)),
        ],
        out_specs=pl.BlockSpec((_TC_ROWS,), lambda i: (i,)),
        out_shape=jax.ShapeDtypeStruct((BATCH,), jnp.float32),
    )(t, u_gath, i_gath)


def kernel(user_ids, item_ids, item_text_feats, user_emb, item_id_emb,
           W_text, b_text):
    t = _text_proj(item_text_feats, W_text, b_text.reshape(1, ID_DIM))
    u_gath = _gather_call(OUT_DIM)(user_ids, user_emb)
    i_gath = _gather_call(ID_DIM)(item_ids, item_id_emb)
    return _combine(t, u_gath, i_gath)
